# Initial kernel scaffold; baseline (speedup 1.0000x reference)
import functools

import jax
import jax.numpy as jnp
from jax import lax
from jax.experimental import pallas as pl
from jax.experimental.pallas import tpu as pltpu
from jax.experimental.pallas import tpu_sc as plsc

N = 15135
NP = 15360
B = 4
H = 32
NB = B * H
R4 = NP * B
E = 484320
NCORE = 2
NSUB = 16
NW = NCORE * NSUB
CH = 128
NCHUNK = 119
EW = CH * NCHUNK
EP = NW * EW
RT = NP // NSUB
ZR = 120

_sc_mesh = plsc.VectorSubcoreMesh(core_axis_name="c", subcore_axis_name="s")


def _elu(v):
    return jnp.where(v > 0, v, jnp.exp(jnp.minimum(v, 0.0)) - 1.0)


@functools.partial(
    pl.kernel,
    out_type=jax.ShapeDtypeStruct((NW, NP), jnp.float32),
    mesh=_sc_mesh,
    scratch_types=[
        pltpu.VMEM((CH,), jnp.int32),
        pltpu.VMEM((NP,), jnp.float32),
    ],
)
def _sc_hist(dst_hbm, hist_hbm, idx_v, hist_v):
    c = lax.axis_index("c")
    s = lax.axis_index("s")
    w = c * NSUB + s

    def zero_body(i, carry):
        hist_v[pl.ds(pl.multiple_of(i * 16, 16), 16)] = jnp.zeros((16,), jnp.float32)
        return carry

    lax.fori_loop(0, NP // 16, zero_body, 0)

    ones = jnp.ones((16,), jnp.float32)
    ebase = w * EW

    def chunk_body(i, carry):
        off = pl.multiple_of(ebase + i * CH, 8)
        pltpu.sync_copy(dst_hbm.at[pl.ds(off, CH)], idx_v)

        def group_body(j, inner):
            idx16 = idx_v[pl.ds(pl.multiple_of(j * 16, 16), 16)]
            plsc.addupdate_scatter(hist_v, [idx16], ones)
            return inner

        lax.fori_loop(0, CH // 16, group_body, 0)
        return carry

    lax.fori_loop(0, NCHUNK, chunk_body, 0)
    pltpu.sync_copy(hist_v, hist_hbm.at[w])


@functools.partial(
    pl.kernel,
    out_type=jax.ShapeDtypeStruct((NCORE, NP, NB), jnp.float32),
    mesh=_sc_mesh,
    scratch_types=[
        pltpu.VMEM((CH,), jnp.int32),
        pltpu.VMEM((CH,), jnp.int32),
        pltpu.VMEM((CH, NB), jnp.float32),
        pltpu.VMEM((ZR, NB), jnp.float32),
        pltpu.VMEM_SHARED((NP, NB), jnp.float32),
        pltpu.SemaphoreType.DMA,
    ],
)
def _sc_scatter(g_hbm, src_hbm, dst_hbm, part_hbm,
                idxs_v, idxd_v, rows_v, zero_v, acc_sh, gsem):
    c = lax.axis_index("c")
    s = lax.axis_index("s")
    w = c * NSUB + s

    def zfill(i, carry):
        zero_v[i // 8, pl.ds(pl.multiple_of((i % 8) * 16, 16), 16)] = (
            jnp.zeros((16,), jnp.float32))
        return carry

    lax.fori_loop(0, ZR * 8, zfill, 0)

    def zcopy(j, carry):
        pltpu.sync_copy(zero_v, acc_sh.at[pl.ds(s * RT + j * ZR, ZR)])
        return carry

    lax.fori_loop(0, RT // ZR, zcopy, 0)
    plsc.subcore_barrier()

    ebase = w * EW

    def chunk_body(i, carry):
        off = pl.multiple_of(ebase + i * CH, 8)
        pltpu.sync_copy(src_hbm.at[pl.ds(off, CH)], idxs_v)
        pltpu.sync_copy(dst_hbm.at[pl.ds(off, CH)], idxd_v)
        pltpu.async_copy(g_hbm.at[idxs_v], rows_v, gsem).wait()
        pltpu.sync_copy(rows_v, acc_sh.at[idxd_v], add=True)
        return carry

    lax.fori_loop(0, NCHUNK, chunk_body, 0)
    plsc.subcore_barrier()

    ro = pl.multiple_of(s * RT, 8)
    pltpu.sync_copy(acc_sh.at[pl.ds(ro, RT)], part_hbm.at[c, pl.ds(ro, RT)])


_DKB = 1536


def _dinv_body(hist_ref, out_ref):
    deg = jnp.sum(hist_ref[...], axis=0, keepdims=True) + 1.0
    out_ref[...] = lax.rsqrt(deg)


_dinv_call = pl.pallas_call(
    _dinv_body,
    grid=(NP // _DKB,),
    in_specs=[pl.BlockSpec((NW, _DKB), lambda i: (0, i))],
    out_specs=pl.BlockSpec((1, _DKB), lambda i: (0, i)),
    out_shape=jax.ShapeDtypeStruct((1, NP), jnp.float32),
)

_RB = 1024


def _mm_scale_body(x_ref, w_ref, d_ref, o_ref):
    o_ref[...] = d_ref[...] * jnp.dot(
        x_ref[...], w_ref[...], preferred_element_type=jnp.float32)


_mm_scale_call = pl.pallas_call(
    _mm_scale_body,
    grid=(R4 // _RB,),
    in_specs=[
        pl.BlockSpec((_RB, 128), lambda i: (i, 0)),
        pl.BlockSpec((128, H), lambda i: (0, 0)),
        pl.BlockSpec((_RB, 1), lambda i: (i, 0)),
    ],
    out_specs=pl.BlockSpec((_RB, H), lambda i: (i, 0)),
    out_shape=jax.ShapeDtypeStruct((R4, H), jnp.float32),
)


def _comb_body(p_ref, g_ref, d_ref, b_ref, w_ref, h_ref, gn_ref):
    d = d_ref[...]
    h = _elu(d * (p_ref[0] + p_ref[1] + g_ref[...]) + b_ref[...])
    h_ref[...] = h
    gn_ref[...] = d * jnp.dot(h, w_ref[...], preferred_element_type=jnp.float32)


_comb_call = pl.pallas_call(
    _comb_body,
    grid=(R4 // _RB,),
    in_specs=[
        pl.BlockSpec((NCORE, _RB, H), lambda i: (0, i, 0)),
        pl.BlockSpec((_RB, H), lambda i: (i, 0)),
        pl.BlockSpec((_RB, 1), lambda i: (i, 0)),
        pl.BlockSpec((1, H), lambda i: (0, 0)),
        pl.BlockSpec((H, H), lambda i: (0, 0)),
    ],
    out_specs=[
        pl.BlockSpec((_RB, H), lambda i: (i, 0)),
        pl.BlockSpec((_RB, H), lambda i: (i, 0)),
    ],
    out_shape=[
        jax.ShapeDtypeStruct((R4, H), jnp.float32),
        jax.ShapeDtypeStruct((R4, H), jnp.float32),
    ],
)


def _xg_body(p_ref, g_ref, d_ref, b_ref, h1_ref, h2_ref, wf_ref, bf_ref, xg_ref):
    d = d_ref[...]
    h3 = _elu(d * (p_ref[0] + p_ref[1] + g_ref[...]) + b_ref[...])
    wf = wf_ref[...]
    xg_ref[...] = (
        jnp.dot(h1_ref[...], wf[:, 0:1], preferred_element_type=jnp.float32)
        + jnp.dot(h2_ref[...], wf[:, 1:2], preferred_element_type=jnp.float32)
        + jnp.dot(h3, wf[:, 2:3], preferred_element_type=jnp.float32)
        + bf_ref[...])


_xg_call = pl.pallas_call(
    _xg_body,
    grid=(R4 // _RB,),
    in_specs=[
        pl.BlockSpec((NCORE, _RB, H), lambda i: (0, i, 0)),
        pl.BlockSpec((_RB, H), lambda i: (i, 0)),
        pl.BlockSpec((_RB, 1), lambda i: (i, 0)),
        pl.BlockSpec((1, H), lambda i: (0, 0)),
        pl.BlockSpec((_RB, H), lambda i: (i, 0)),
        pl.BlockSpec((_RB, H), lambda i: (i, 0)),
        pl.BlockSpec((H, 3), lambda i: (0, 0)),
        pl.BlockSpec((1, 1), lambda i: (0, 0)),
    ],
    out_specs=pl.BlockSpec((_RB, 1), lambda i: (i, 0)),
    out_shape=jax.ShapeDtypeStruct((R4, 1), jnp.float32),
)

_HKB = 1536
_HSTEPS = NP // _HKB


def _head_body(xg_ref, wl1_ref, bl1_ref, wl2_ref, bl2_ref, out_ref, acc_ref):
    k = pl.program_id(0)

    @pl.when(k == 0)
    def _init():
        acc_ref[...] = jnp.zeros_like(acc_ref)

    acc_ref[...] += jnp.dot(
        xg_ref[...], wl1_ref[...], preferred_element_type=jnp.float32)

    @pl.when(k == _HSTEPS - 1)
    def _fin():
        y = _elu(acc_ref[...] + bl1_ref[...])
        z = jnp.dot(y, wl2_ref[...], preferred_element_type=jnp.float32)
        z = z + bl2_ref[...]
        m = jnp.max(z, axis=-1, keepdims=True)
        ez = jnp.exp(z - m)
        out_ref[...] = (z - m) - jnp.log(jnp.sum(ez, axis=-1, keepdims=True))


_head_call = pl.pallas_call(
    _head_body,
    grid=(_HSTEPS,),
    in_specs=[
        pl.BlockSpec((8, _HKB), lambda k: (0, k)),
        pl.BlockSpec((_HKB, 256), lambda k: (k, 0)),
        pl.BlockSpec((1, 256), lambda k: (0, 0)),
        pl.BlockSpec((256, 128), lambda k: (0, 0)),
        pl.BlockSpec((1, 128), lambda k: (0, 0)),
    ],
    out_specs=pl.BlockSpec((8, 128), lambda k: (0, 0)),
    out_shape=jax.ShapeDtypeStruct((8, 128), jnp.float32),
    scratch_shapes=[pltpu.VMEM((8, 256), jnp.float32)],
)


def kernel(x, batch, edge_index, nodes_graphlets,
           W1, b1, W2, b2, W3, b3, Wfc, bfc, Wl1, bl1, Wl2, bl2):
    f32 = jnp.float32
    G = nodes_graphlets.shape[1]

    xt = jnp.transpose(x, (1, 0, 2))
    pe = jnp.broadcast_to(nodes_graphlets[:, None, :], (N, B, G))
    h0 = jnp.concatenate(
        [xt, pe, jnp.zeros((N, B, 128 - 1 - G), f32)], axis=2)
    h0 = jnp.pad(h0, ((0, NP - N), (0, 0), (0, 0))).reshape(R4, 128)
    W1p = jnp.pad(W1, ((0, 128 - 1 - G), (0, 0)))

    src = jnp.pad(edge_index[0], (0, EP - E), constant_values=N)
    dst = jnp.pad(edge_index[1], (0, EP - E), constant_values=N)

    hist = _sc_hist(dst)
    dinv = _dinv_call(hist)
    d4 = jnp.broadcast_to(dinv.reshape(NP, 1), (NP, B)).reshape(R4, 1)

    g1 = _mm_scale_call(h0, W1p, d4)
    p1 = _sc_scatter(g1.reshape(NP, NB), src, dst).reshape(NCORE, R4, H)
    h1, g2 = _comb_call(p1, g1, d4, b1[None], W2)
    p2 = _sc_scatter(g2.reshape(NP, NB), src, dst).reshape(NCORE, R4, H)
    h2, g3 = _comb_call(p2, g2, d4, b2[None], W3)
    p3 = _sc_scatter(g3.reshape(NP, NB), src, dst).reshape(NCORE, R4, H)

    xg = _xg_call(p3, g3, d4, b3[None], h1, h2,
                  Wfc.reshape(H, 3), bfc.reshape(1, 1))

    xgt = xg.reshape(NP, B)[:N].T
    xg8 = jnp.zeros((8, NP), f32).at[:B, :N].set(xgt)
    Wl1p = jnp.pad(Wl1, ((0, NP - N), (0, 0)))
    Wl2p = jnp.pad(Wl2, ((0, 0), (0, 126)))
    bl2p = jnp.concatenate([bl2, jnp.full((126,), -1e30, f32)])

    out = _head_call(xg8, Wl1p, bl1[None], Wl2p, bl2p[None])
    return out[:B, :2]


# speedup vs baseline: 68.9992x; 68.9992x over previous
import functools

import jax
import jax.numpy as jnp
from jax import lax
from jax.experimental import pallas as pl
from jax.experimental.pallas import tpu as pltpu
from jax.experimental.pallas import tpu_sc as plsc

N = 15135
NP = 15360
B = 4
H = 32
NB = B * H
R4 = NP * B
E = 484320
NCORE = 2
NSUB = 16
NW = NCORE * NSUB
CH = 128
NCHUNK = 119
EW = CH * NCHUNK
EP = NW * EW
RT = NP // NSUB
ZR = 120

def _elu(v):
    return jnp.where(v > 0, v, jnp.exp(jnp.minimum(v, 0.0)) - 1.0)


def _sc_hist_body(dst_hbm, hist_hbm, idx_v, hist_v):
    c = lax.axis_index("c")
    s = lax.axis_index("s")
    w = c * NSUB + s

    def zero_body(i, carry):
        hist_v[pl.ds(pl.multiple_of(i * 16, 16), 16)] = jnp.zeros((16,), jnp.float32)
        return carry

    lax.fori_loop(0, NP // 16, zero_body, 0)

    ones = jnp.ones((16,), jnp.float32)
    ebase = w * EW

    def chunk_body(i, carry):
        off = pl.multiple_of(ebase + i * CH, 8)
        pltpu.sync_copy(dst_hbm.at[pl.ds(off, CH)], idx_v)

        def group_body(j, inner):
            idx16 = idx_v[pl.ds(pl.multiple_of(j * 16, 16), 16)]
            plsc.addupdate_scatter(hist_v, [idx16], ones)
            return inner

        lax.fori_loop(0, CH // 16, group_body, 0)
        return carry

    lax.fori_loop(0, NCHUNK, chunk_body, 0)
    pltpu.sync_copy(hist_v, hist_hbm.at[w])


def _sc_scatter_body(g_hbm, src_hbm, dst_hbm, part_hbm,
                     idxs_v, idxd_v, rows_v, zero_v, acc_sh, gsem):
    c = lax.axis_index("c")
    s = lax.axis_index("s")
    w = c * NSUB + s

    def zfill(i, carry):
        zero_v[i // 8, pl.ds(pl.multiple_of((i % 8) * 16, 16), 16)] = (
            jnp.zeros((16,), jnp.float32))
        return carry

    lax.fori_loop(0, ZR * 8, zfill, 0)

    ebase = w * EW
    for half in range(2):
        def zcopy(j, carry):
            pltpu.sync_copy(zero_v, acc_sh.at[pl.ds(s * RT + j * ZR, ZR)])
            return carry

        lax.fori_loop(0, RT // ZR, zcopy, 0)
        plsc.subcore_barrier()

        g_half = g_hbm.at[half]

        def chunk_body(i, carry):
            off = pl.multiple_of(ebase + i * CH, 8)
            pltpu.sync_copy(src_hbm.at[pl.ds(off, CH)], idxs_v)
            pltpu.sync_copy(dst_hbm.at[pl.ds(off, CH)], idxd_v)
            pltpu.async_copy(g_half.at[idxs_v], rows_v, gsem).wait()
            pltpu.sync_copy(rows_v, acc_sh.at[idxd_v], add=True)
            return carry

        lax.fori_loop(0, NCHUNK, chunk_body, 0)
        plsc.subcore_barrier()

        ro = pl.multiple_of(s * RT, 8)
        pltpu.sync_copy(acc_sh.at[pl.ds(ro, RT)],
                        part_hbm.at[c, half, pl.ds(ro, RT)])
        plsc.subcore_barrier()


@functools.lru_cache(maxsize=1)
def _sc_kernels():
    mesh = plsc.VectorSubcoreMesh(
        core_axis_name="c", subcore_axis_name="s",
        num_cores=NCORE, num_subcores=NSUB)
    params = pltpu.CompilerParams(
        needs_layout_passes=False, use_tc_tiling_on_sc=False)
    sc_hist = pl.kernel(
        _sc_hist_body,
        out_type=jax.ShapeDtypeStruct((NW, NP), jnp.float32),
        mesh=mesh,
        compiler_params=params,
        scratch_types=[
            pltpu.VMEM((CH,), jnp.int32),
            pltpu.VMEM((NP,), jnp.float32),
        ],
    )
    sc_scatter = pl.kernel(
        _sc_scatter_body,
        out_type=jax.ShapeDtypeStruct((NCORE, 2, NP, NB // 2), jnp.float32),
        mesh=mesh,
        compiler_params=params,
        scratch_types=[
            pltpu.VMEM((CH,), jnp.int32),
            pltpu.VMEM((CH,), jnp.int32),
            pltpu.VMEM((CH, NB // 2), jnp.float32),
            pltpu.VMEM((ZR, NB // 2), jnp.float32),
            pltpu.VMEM_SHARED((NP, NB // 2), jnp.float32),
            pltpu.SemaphoreType.DMA,
        ],
    )
    return sc_hist, sc_scatter


_DKB = 1536


def _dinv_body(hist_ref, out_ref):
    deg = jnp.sum(hist_ref[...], axis=0, keepdims=True) + 1.0
    out_ref[...] = lax.rsqrt(deg)


_dinv_call = pl.pallas_call(
    _dinv_body,
    grid=(NP // _DKB,),
    in_specs=[pl.BlockSpec((NW, _DKB), lambda i: (0, i))],
    out_specs=pl.BlockSpec((1, _DKB), lambda i: (0, i)),
    out_shape=jax.ShapeDtypeStruct((1, NP), jnp.float32),
)

_RB = 1024


def _mm_scale_body(x_ref, w_ref, d_ref, o_ref):
    o_ref[...] = d_ref[...] * jnp.dot(
        x_ref[...], w_ref[...], preferred_element_type=jnp.float32)


_mm_scale_call = pl.pallas_call(
    _mm_scale_body,
    grid=(R4 // _RB,),
    in_specs=[
        pl.BlockSpec((_RB, 128), lambda i: (i, 0)),
        pl.BlockSpec((128, H), lambda i: (0, 0)),
        pl.BlockSpec((_RB, 1), lambda i: (i, 0)),
    ],
    out_specs=pl.BlockSpec((_RB, H), lambda i: (i, 0)),
    out_shape=jax.ShapeDtypeStruct((R4, H), jnp.float32),
)


def _comb_body(p_ref, g_ref, d_ref, b_ref, w_ref, h_ref, gn_ref):
    d = d_ref[...]
    h = _elu(d * (p_ref[0] + p_ref[1] + g_ref[...]) + b_ref[...])
    h_ref[...] = h
    gn_ref[...] = d * jnp.dot(h, w_ref[...], preferred_element_type=jnp.float32)


_comb_call = pl.pallas_call(
    _comb_body,
    grid=(R4 // _RB,),
    in_specs=[
        pl.BlockSpec((NCORE, _RB, H), lambda i: (0, i, 0)),
        pl.BlockSpec((_RB, H), lambda i: (i, 0)),
        pl.BlockSpec((_RB, 1), lambda i: (i, 0)),
        pl.BlockSpec((1, H), lambda i: (0, 0)),
        pl.BlockSpec((H, H), lambda i: (0, 0)),
    ],
    out_specs=[
        pl.BlockSpec((_RB, H), lambda i: (i, 0)),
        pl.BlockSpec((_RB, H), lambda i: (i, 0)),
    ],
    out_shape=[
        jax.ShapeDtypeStruct((R4, H), jnp.float32),
        jax.ShapeDtypeStruct((R4, H), jnp.float32),
    ],
)


def _xg_body(p_ref, g_ref, d_ref, b_ref, h1_ref, h2_ref, wf_ref, bf_ref, xg_ref):
    d = d_ref[...]
    h3 = _elu(d * (p_ref[0] + p_ref[1] + g_ref[...]) + b_ref[...])
    wf = wf_ref[...]
    xg_ref[...] = (
        jnp.dot(h1_ref[...], wf[:, 0:1], preferred_element_type=jnp.float32)
        + jnp.dot(h2_ref[...], wf[:, 1:2], preferred_element_type=jnp.float32)
        + jnp.dot(h3, wf[:, 2:3], preferred_element_type=jnp.float32)
        + bf_ref[...])


_xg_call = pl.pallas_call(
    _xg_body,
    grid=(R4 // _RB,),
    in_specs=[
        pl.BlockSpec((NCORE, _RB, H), lambda i: (0, i, 0)),
        pl.BlockSpec((_RB, H), lambda i: (i, 0)),
        pl.BlockSpec((_RB, 1), lambda i: (i, 0)),
        pl.BlockSpec((1, H), lambda i: (0, 0)),
        pl.BlockSpec((_RB, H), lambda i: (i, 0)),
        pl.BlockSpec((_RB, H), lambda i: (i, 0)),
        pl.BlockSpec((H, 3), lambda i: (0, 0)),
        pl.BlockSpec((1, 1), lambda i: (0, 0)),
    ],
    out_specs=pl.BlockSpec((_RB, 1), lambda i: (i, 0)),
    out_shape=jax.ShapeDtypeStruct((R4, 1), jnp.float32),
)

_HKB = 1536
_HSTEPS = NP // _HKB


def _head_body(xg_ref, wl1_ref, bl1_ref, wl2_ref, bl2_ref, out_ref, acc_ref):
    k = pl.program_id(0)

    @pl.when(k == 0)
    def _init():
        acc_ref[...] = jnp.zeros_like(acc_ref)

    acc_ref[...] += jnp.dot(
        xg_ref[...], wl1_ref[...], preferred_element_type=jnp.float32)

    @pl.when(k == _HSTEPS - 1)
    def _fin():
        y = _elu(acc_ref[...] + bl1_ref[...])
        z = jnp.dot(y, wl2_ref[...], preferred_element_type=jnp.float32)
        z = z + bl2_ref[...]
        m = jnp.max(z, axis=-1, keepdims=True)
        ez = jnp.exp(z - m)
        out_ref[...] = (z - m) - jnp.log(jnp.sum(ez, axis=-1, keepdims=True))


_head_call = pl.pallas_call(
    _head_body,
    grid=(_HSTEPS,),
    in_specs=[
        pl.BlockSpec((8, _HKB), lambda k: (0, k)),
        pl.BlockSpec((_HKB, 256), lambda k: (k, 0)),
        pl.BlockSpec((1, 256), lambda k: (0, 0)),
        pl.BlockSpec((256, 128), lambda k: (0, 0)),
        pl.BlockSpec((1, 128), lambda k: (0, 0)),
    ],
    out_specs=pl.BlockSpec((8, 128), lambda k: (0, 0)),
    out_shape=jax.ShapeDtypeStruct((8, 128), jnp.float32),
    scratch_shapes=[pltpu.VMEM((8, 256), jnp.float32)],
)


def kernel(x, batch, edge_index, nodes_graphlets,
           W1, b1, W2, b2, W3, b3, Wfc, bfc, Wl1, bl1, Wl2, bl2):
    f32 = jnp.float32
    G = nodes_graphlets.shape[1]

    xt = x.reshape(B, N).T
    xpb = xt.reshape(N, 2, 2).transpose(1, 0, 2)
    pe = jnp.broadcast_to(nodes_graphlets[None, :, None, :], (2, N, 2, G))
    h0 = jnp.concatenate(
        [xpb[..., None], pe, jnp.zeros((2, N, 2, 128 - 1 - G), f32)], axis=3)
    h0 = jnp.pad(h0, ((0, 0), (0, NP - N), (0, 0), (0, 0))).reshape(R4, 128)
    W1p = jnp.pad(W1, ((0, 128 - 1 - G), (0, 0)))

    src = jnp.pad(edge_index[0], (0, EP - E), constant_values=N)
    dst = jnp.pad(edge_index[1], (0, EP - E), constant_values=N)

    _sc_hist, _sc_scatter = _sc_kernels()
    hist = _sc_hist(dst)
    dinv = _dinv_call(hist)
    dnb = jnp.broadcast_to(dinv.reshape(NP, 1), (NP, 2)).reshape(NP * 2, 1)
    d4 = jnp.concatenate([dnb, dnb], axis=0)

    def scat(g):
        return _sc_scatter(
            g.reshape(2, NP, NB // 2), src, dst).reshape(NCORE, R4, H)

    g1 = _mm_scale_call(h0, W1p, d4)
    p1 = scat(g1)
    h1, g2 = _comb_call(p1, g1, d4, b1[None], W2)
    p2 = scat(g2)
    h2, g3 = _comb_call(p2, g2, d4, b2[None], W3)
    p3 = scat(g3)

    xg = _xg_call(p3, g3, d4, b3[None], h1, h2,
                  Wfc.reshape(H, 3), bfc.reshape(1, 1))

    xgt = xg.reshape(2, NP, 2).transpose(1, 0, 2).reshape(NP, B)[:N].T
    xg8 = jnp.zeros((8, NP), f32).at[:B, :N].set(xgt)
    Wl1p = jnp.pad(Wl1, ((0, NP - N), (0, 0)))
    Wl2p = jnp.pad(Wl2, ((0, 0), (0, 126)))
    bl2p = jnp.concatenate([bl2, jnp.full((126,), -1e30, f32)])

    out = _head_call(xg8, Wl1p, bl1[None], Wl2p, bl2p[None])
    return out[:B, :2]


# speedup vs baseline: 77.1960x; 1.1188x over previous
import functools

import jax
import jax.numpy as jnp
from jax import lax
from jax.experimental import pallas as pl
from jax.experimental.pallas import tpu as pltpu
from jax.experimental.pallas import tpu_sc as plsc

N = 15135
NP = 15360
B = 4
H = 32
NB = B * H
R4 = NP * B
E = 484320
NCORE = 2
NSUB = 16
NW = NCORE * NSUB
CH = 128
NCHUNK = 120
EW = CH * NCHUNK
EP = NW * EW
RT = NP // NSUB
ZR = 60
NBUF = 4
NGRP = NCHUNK // NBUF

def _elu(v):
    return jnp.where(v > 0, v, jnp.exp(jnp.minimum(v, 0.0)) - 1.0)


def _sc_hist_body(dst_hbm, hist_hbm, idx_v, hist_v):
    c = lax.axis_index("c")
    s = lax.axis_index("s")
    w = c * NSUB + s

    pltpu.sync_copy(dst_hbm.at[pl.ds(w * NCHUNK, NCHUNK)], idx_v)

    def zero_body(i, carry):
        hist_v[pl.ds(pl.multiple_of(i * 16, 16), 16)] = jnp.zeros((16,), jnp.float32)
        return carry

    lax.fori_loop(0, NP // 16, zero_body, 0)

    ones = jnp.ones((16,), jnp.float32)

    def group_body(j, carry):
        idx16 = idx_v[j // 8, pl.ds(pl.multiple_of((j % 8) * 16, 16), 16)]
        plsc.addupdate_scatter(hist_v, [idx16], ones)
        return carry

    lax.fori_loop(0, NCHUNK * (CH // 16), group_body, 0)
    pltpu.sync_copy(hist_v, hist_hbm.at[w])


def _sc_scatter_body(g_hbm, src_hbm, dst_hbm, part_hbm,
                     idxs_v, idxd_v, rows_v, zero_v, acc_sh, *sems):
    gsems = sems[:NBUF]
    ssems = sems[NBUF:]
    c = lax.axis_index("c")
    s = lax.axis_index("s")
    w = c * NSUB + s

    pltpu.sync_copy(src_hbm.at[pl.ds(w * NCHUNK, NCHUNK)], idxs_v)
    pltpu.sync_copy(dst_hbm.at[pl.ds(w * NCHUNK, NCHUNK)], idxd_v)

    def zfill(i, carry):
        zero_v[i // 4, pl.ds(pl.multiple_of((i % 4) * 16, 16), 16)] = (
            jnp.zeros((16,), jnp.float32))
        return carry

    lax.fori_loop(0, ZR * 4, zfill, 0)

    for half in range(2):
        def zcopy(j, carry):
            pltpu.sync_copy(zero_v, acc_sh.at[pl.ds(s * RT + j * ZR, ZR)])
            return carry

        lax.fori_loop(0, RT // ZR, zcopy, 0)
        plsc.subcore_barrier()

        g_half = g_hbm.at[half]

        def group_body(gidx, carry):
            base = gidx * NBUF
            gds = [
                pltpu.async_copy(
                    g_half.at[idxs_v.at[base + b]], rows_v.at[b], gsems[b])
                for b in range(NBUF)
            ]
            sds = []
            for b in range(NBUF):
                gds[b].wait()
                sds.append(pltpu.async_copy(
                    rows_v.at[b], acc_sh.at[idxd_v.at[base + b]],
                    ssems[b], add=True))
            for b in range(NBUF):
                sds[b].wait()
            return carry

        lax.fori_loop(0, NGRP, group_body, 0)
        plsc.subcore_barrier()

        ro = pl.multiple_of(s * RT, 8)
        pltpu.sync_copy(acc_sh.at[pl.ds(ro, RT)],
                        part_hbm.at[c, half, pl.ds(ro, RT)])
        plsc.subcore_barrier()


@functools.lru_cache(maxsize=1)
def _sc_kernels():
    mesh = plsc.VectorSubcoreMesh(
        core_axis_name="c", subcore_axis_name="s",
        num_cores=NCORE, num_subcores=NSUB)
    params = pltpu.CompilerParams(
        needs_layout_passes=False, use_tc_tiling_on_sc=False)
    sc_hist = pl.kernel(
        _sc_hist_body,
        out_type=jax.ShapeDtypeStruct((NW, NP), jnp.float32),
        mesh=mesh,
        compiler_params=params,
        scratch_types=[
            pltpu.VMEM((NCHUNK, CH), jnp.int32),
            pltpu.VMEM((NP,), jnp.float32),
        ],
    )
    sc_scatter = pl.kernel(
        _sc_scatter_body,
        out_type=jax.ShapeDtypeStruct((NCORE, 2, NP, NB // 2), jnp.float32),
        mesh=mesh,
        compiler_params=params,
        scratch_types=(
            [
                pltpu.VMEM((NCHUNK, CH), jnp.int32),
                pltpu.VMEM((NCHUNK, CH), jnp.int32),
                pltpu.VMEM((NBUF, CH, NB // 2), jnp.float32),
                pltpu.VMEM((ZR, NB // 2), jnp.float32),
                pltpu.VMEM_SHARED((NP, NB // 2), jnp.float32),
            ]
            + [pltpu.SemaphoreType.DMA] * (2 * NBUF)
        ),
    )
    return sc_hist, sc_scatter


_DKB = 1536


def _dinv_body(hist_ref, out_ref):
    deg = jnp.sum(hist_ref[...], axis=0, keepdims=True) + 1.0
    out_ref[...] = lax.rsqrt(deg)


_dinv_call = pl.pallas_call(
    _dinv_body,
    grid=(NP // _DKB,),
    in_specs=[pl.BlockSpec((NW, _DKB), lambda i: (0, i))],
    out_specs=pl.BlockSpec((1, _DKB), lambda i: (0, i)),
    out_shape=jax.ShapeDtypeStruct((1, NP), jnp.float32),
)

_RB = 1024


def _mm_scale_body(x_ref, w_ref, d_ref, o_ref):
    o_ref[...] = d_ref[...] * jnp.dot(
        x_ref[...], w_ref[...], preferred_element_type=jnp.float32)


_mm_scale_call = pl.pallas_call(
    _mm_scale_body,
    grid=(R4 // _RB,),
    in_specs=[
        pl.BlockSpec((_RB, 128), lambda i: (i, 0)),
        pl.BlockSpec((128, H), lambda i: (0, 0)),
        pl.BlockSpec((_RB, 1), lambda i: (i, 0)),
    ],
    out_specs=pl.BlockSpec((_RB, H), lambda i: (i, 0)),
    out_shape=jax.ShapeDtypeStruct((R4, H), jnp.float32),
)


def _comb_body(p_ref, g_ref, d_ref, b_ref, w_ref, h_ref, gn_ref):
    d = d_ref[...]
    h = _elu(d * (p_ref[0] + p_ref[1] + g_ref[...]) + b_ref[...])
    h_ref[...] = h
    gn_ref[...] = d * jnp.dot(h, w_ref[...], preferred_element_type=jnp.float32)


_comb_call = pl.pallas_call(
    _comb_body,
    grid=(R4 // _RB,),
    in_specs=[
        pl.BlockSpec((NCORE, _RB, H), lambda i: (0, i, 0)),
        pl.BlockSpec((_RB, H), lambda i: (i, 0)),
        pl.BlockSpec((_RB, 1), lambda i: (i, 0)),
        pl.BlockSpec((1, H), lambda i: (0, 0)),
        pl.BlockSpec((H, H), lambda i: (0, 0)),
    ],
    out_specs=[
        pl.BlockSpec((_RB, H), lambda i: (i, 0)),
        pl.BlockSpec((_RB, H), lambda i: (i, 0)),
    ],
    out_shape=[
        jax.ShapeDtypeStruct((R4, H), jnp.float32),
        jax.ShapeDtypeStruct((R4, H), jnp.float32),
    ],
)


def _xg_body(p_ref, g_ref, d_ref, b_ref, h1_ref, h2_ref, wf_ref, bf_ref, xg_ref):
    d = d_ref[...]
    h3 = _elu(d * (p_ref[0] + p_ref[1] + g_ref[...]) + b_ref[...])
    wf = wf_ref[...]
    xg_ref[...] = (
        jnp.dot(h1_ref[...], wf[:, 0:1], preferred_element_type=jnp.float32)
        + jnp.dot(h2_ref[...], wf[:, 1:2], preferred_element_type=jnp.float32)
        + jnp.dot(h3, wf[:, 2:3], preferred_element_type=jnp.float32)
        + bf_ref[...])


_xg_call = pl.pallas_call(
    _xg_body,
    grid=(R4 // _RB,),
    in_specs=[
        pl.BlockSpec((NCORE, _RB, H), lambda i: (0, i, 0)),
        pl.BlockSpec((_RB, H), lambda i: (i, 0)),
        pl.BlockSpec((_RB, 1), lambda i: (i, 0)),
        pl.BlockSpec((1, H), lambda i: (0, 0)),
        pl.BlockSpec((_RB, H), lambda i: (i, 0)),
        pl.BlockSpec((_RB, H), lambda i: (i, 0)),
        pl.BlockSpec((H, 3), lambda i: (0, 0)),
        pl.BlockSpec((1, 1), lambda i: (0, 0)),
    ],
    out_specs=pl.BlockSpec((_RB, 1), lambda i: (i, 0)),
    out_shape=jax.ShapeDtypeStruct((R4, 1), jnp.float32),
)

_HKB = 1536
_HSTEPS = NP // _HKB


def _head_body(xg_ref, wl1_ref, bl1_ref, wl2_ref, bl2_ref, out_ref, acc_ref):
    k = pl.program_id(0)

    @pl.when(k == 0)
    def _init():
        acc_ref[...] = jnp.zeros_like(acc_ref)

    acc_ref[...] += jnp.dot(
        xg_ref[...], wl1_ref[...], preferred_element_type=jnp.float32)

    @pl.when(k == _HSTEPS - 1)
    def _fin():
        y = _elu(acc_ref[...] + bl1_ref[...])
        z = jnp.dot(y, wl2_ref[...], preferred_element_type=jnp.float32)
        z = z + bl2_ref[...]
        m = jnp.max(z, axis=-1, keepdims=True)
        ez = jnp.exp(z - m)
        out_ref[...] = (z - m) - jnp.log(jnp.sum(ez, axis=-1, keepdims=True))


_head_call = pl.pallas_call(
    _head_body,
    grid=(_HSTEPS,),
    in_specs=[
        pl.BlockSpec((8, _HKB), lambda k: (0, k)),
        pl.BlockSpec((_HKB, 256), lambda k: (k, 0)),
        pl.BlockSpec((1, 256), lambda k: (0, 0)),
        pl.BlockSpec((256, 128), lambda k: (0, 0)),
        pl.BlockSpec((1, 128), lambda k: (0, 0)),
    ],
    out_specs=pl.BlockSpec((8, 128), lambda k: (0, 0)),
    out_shape=jax.ShapeDtypeStruct((8, 128), jnp.float32),
    scratch_shapes=[pltpu.VMEM((8, 256), jnp.float32)],
)


def kernel(x, batch, edge_index, nodes_graphlets,
           W1, b1, W2, b2, W3, b3, Wfc, bfc, Wl1, bl1, Wl2, bl2):
    f32 = jnp.float32
    G = nodes_graphlets.shape[1]

    xt = x.reshape(B, N).T
    xpb = xt.reshape(N, 2, 2).transpose(1, 0, 2)
    pe = jnp.broadcast_to(nodes_graphlets[None, :, None, :], (2, N, 2, G))
    h0 = jnp.concatenate(
        [xpb[..., None], pe, jnp.zeros((2, N, 2, 128 - 1 - G), f32)], axis=3)
    h0 = jnp.pad(h0, ((0, 0), (0, NP - N), (0, 0), (0, 0))).reshape(R4, 128)
    W1p = jnp.pad(W1, ((0, 128 - 1 - G), (0, 0)))

    src = jnp.pad(edge_index[0], (0, EP - E),
                  constant_values=N).reshape(NW * NCHUNK, CH)
    dst = jnp.pad(edge_index[1], (0, EP - E),
                  constant_values=N).reshape(NW * NCHUNK, CH)

    _sc_hist, _sc_scatter = _sc_kernels()
    hist = _sc_hist(dst)
    dinv = _dinv_call(hist)
    dnb = jnp.broadcast_to(dinv.reshape(NP, 1), (NP, 2)).reshape(NP * 2, 1)
    d4 = jnp.concatenate([dnb, dnb], axis=0)

    def scat(g):
        return _sc_scatter(
            g.reshape(2, NP, NB // 2), src, dst).reshape(NCORE, R4, H)

    g1 = _mm_scale_call(h0, W1p, d4)
    p1 = scat(g1)
    h1, g2 = _comb_call(p1, g1, d4, b1[None], W2)
    p2 = scat(g2)
    h2, g3 = _comb_call(p2, g2, d4, b2[None], W3)
    p3 = scat(g3)

    xg = _xg_call(p3, g3, d4, b3[None], h1, h2,
                  Wfc.reshape(H, 3), bfc.reshape(1, 1))

    xgt = xg.reshape(2, NP, 2).transpose(1, 0, 2).reshape(NP, B)[:N].T
    xg8 = jnp.zeros((8, NP), f32).at[:B, :N].set(xgt)
    Wl1p = jnp.pad(Wl1, ((0, NP - N), (0, 0)))
    Wl2p = jnp.pad(Wl2, ((0, 0), (0, 126)))
    bl2p = jnp.concatenate([bl2, jnp.full((126,), -1e30, f32)])

    out = _head_call(xg8, Wl1p, bl1[None], Wl2p, bl2p[None])
    return out[:B, :2]


# speedup vs baseline: 85.1861x; 1.1035x over previous
import functools

import jax
import jax.numpy as jnp
from jax import lax
from jax.experimental import pallas as pl
from jax.experimental.pallas import tpu as pltpu
from jax.experimental.pallas import tpu_sc as plsc

N = 15135
NP = 15360
B = 4
H = 32
NB = B * H
R4 = NP * B
E = 484320
NCORE = 2
NSUB = 16
NW = NCORE * NSUB
CH = 128
NCHUNK = 120
EW = CH * NCHUNK
EP = NW * EW
RT = NP // NSUB
ZR = 60
NBUF = 4
NGRP = NCHUNK // NBUF

def _elu(v):
    return jnp.where(v > 0, v, jnp.exp(jnp.minimum(v, 0.0)) - 1.0)


def _sc_hist_body(dst_hbm, hist_hbm, idx_v, hist_v):
    c = lax.axis_index("c")
    s = lax.axis_index("s")
    w = c * NSUB + s

    pltpu.sync_copy(dst_hbm.at[pl.ds(w * NCHUNK, NCHUNK)], idx_v)

    zeros16 = jnp.zeros((16,), jnp.float32)

    def zero_body(i, carry):
        for j in range(8):
            hist_v[pl.ds(pl.multiple_of(i * 128 + j * 16, 16), 16)] = zeros16
        return carry

    lax.fori_loop(0, NP // 128, zero_body, 0)

    ones = jnp.ones((16,), jnp.float32)

    def chunk_body(i, carry):
        for j in range(CH // 16):
            idx16 = idx_v[i, pl.ds(pl.multiple_of(j * 16, 16), 16)]
            plsc.addupdate_scatter(hist_v, [idx16], ones)
        return carry

    lax.fori_loop(0, NCHUNK, chunk_body, 0)
    pltpu.sync_copy(hist_v, hist_hbm.at[w])


def _sc_scatter_body(g_hbm, src_hbm, dst_hbm, part_hbm,
                     idxs_v, idxd_v, rows_v, zero_v, acc_sh, *sems):
    gsems = sems[:NBUF]
    ssems = sems[NBUF:]
    c = lax.axis_index("c")
    s = lax.axis_index("s")
    w = c * NSUB + s

    pltpu.sync_copy(src_hbm.at[pl.ds(w * NCHUNK, NCHUNK)], idxs_v)
    pltpu.sync_copy(dst_hbm.at[pl.ds(w * NCHUNK, NCHUNK)], idxd_v)

    def zfill(i, carry):
        zero_v[i // 4, pl.ds(pl.multiple_of((i % 4) * 16, 16), 16)] = (
            jnp.zeros((16,), jnp.float32))
        return carry

    lax.fori_loop(0, ZR * 4, zfill, 0)

    for half in range(2):
        def zcopy(j, carry):
            pltpu.sync_copy(zero_v, acc_sh.at[pl.ds(s * RT + j * ZR, ZR)])
            return carry

        lax.fori_loop(0, RT // ZR, zcopy, 0)
        plsc.subcore_barrier()

        g_half = g_hbm.at[half]

        def gather(ci, b):
            pltpu.async_copy(
                g_half.at[idxs_v.at[ci]], rows_v.at[b], gsems[b])

        def gwait(b):
            pltpu.make_async_copy(
                g_half.at[idxs_v.at[0]], rows_v.at[b], gsems[b]).wait()

        def scat(ci, b):
            pltpu.async_copy(
                rows_v.at[b], acc_sh.at[idxd_v.at[ci]], ssems[b], add=True)

        def swait(b):
            pltpu.make_async_copy(
                rows_v.at[b], acc_sh.at[idxd_v.at[0]], ssems[b]).wait()

        for b in range(NBUF):
            gather(b, b)

        def group_body(m, carry):
            base = m * NBUF
            gwait(0); scat(base + 0, 0)
            gwait(1); scat(base + 1, 1)
            swait(0); gather(base + 4, 0)
            swait(1); gather(base + 5, 1)
            gwait(2); scat(base + 2, 2)
            gwait(3); scat(base + 3, 3)
            swait(2); gather(base + 6, 2)
            swait(3); gather(base + 7, 3)
            return carry

        lax.fori_loop(0, NGRP - 1, group_body, 0)
        for b in range(NBUF):
            gwait(b)
            scat(NCHUNK - NBUF + b, b)
        for b in range(NBUF):
            swait(b)
        plsc.subcore_barrier()

        ro = pl.multiple_of(s * RT, 8)
        pltpu.sync_copy(acc_sh.at[pl.ds(ro, RT)],
                        part_hbm.at[c, half, pl.ds(ro, RT)])
        plsc.subcore_barrier()


@functools.lru_cache(maxsize=1)
def _sc_kernels():
    mesh = plsc.VectorSubcoreMesh(
        core_axis_name="c", subcore_axis_name="s",
        num_cores=NCORE, num_subcores=NSUB)
    params = pltpu.CompilerParams(
        needs_layout_passes=False, use_tc_tiling_on_sc=False)
    sc_hist = pl.kernel(
        _sc_hist_body,
        out_type=jax.ShapeDtypeStruct((NW, NP), jnp.float32),
        mesh=mesh,
        compiler_params=params,
        scratch_types=[
            pltpu.VMEM((NCHUNK, CH), jnp.int32),
            pltpu.VMEM((NP,), jnp.float32),
        ],
    )
    sc_scatter = pl.kernel(
        _sc_scatter_body,
        out_type=jax.ShapeDtypeStruct((NCORE, 2, NP, NB // 2), jnp.float32),
        mesh=mesh,
        compiler_params=params,
        scratch_types=(
            [
                pltpu.VMEM((NCHUNK, CH), jnp.int32),
                pltpu.VMEM((NCHUNK, CH), jnp.int32),
                pltpu.VMEM((NBUF, CH, NB // 2), jnp.float32),
                pltpu.VMEM((ZR, NB // 2), jnp.float32),
                pltpu.VMEM_SHARED((NP, NB // 2), jnp.float32),
            ]
            + [pltpu.SemaphoreType.DMA] * (2 * NBUF)
        ),
    )
    return sc_hist, sc_scatter


_DKB = 1536


def _dinv_body(hist_ref, out_ref):
    deg = jnp.sum(hist_ref[...], axis=0, keepdims=True) + 1.0
    out_ref[...] = lax.rsqrt(deg)


_dinv_call = pl.pallas_call(
    _dinv_body,
    grid=(NP // _DKB,),
    in_specs=[pl.BlockSpec((NW, _DKB), lambda i: (0, i))],
    out_specs=pl.BlockSpec((1, _DKB), lambda i: (0, i)),
    out_shape=jax.ShapeDtypeStruct((1, NP), jnp.float32),
)

_RB = 4096


def _mm_body(x_ref, w_ref, o_ref):
    o_ref[...] = jnp.dot(
        x_ref[...], w_ref[...], preferred_element_type=jnp.float32)


_mm_call = pl.pallas_call(
    _mm_body,
    grid=(R4 // _RB,),
    in_specs=[
        pl.BlockSpec((_RB, 128), lambda i: (i, 0)),
        pl.BlockSpec((128, H), lambda i: (0, 0)),
    ],
    out_specs=pl.BlockSpec((_RB, H), lambda i: (i, 0)),
    out_shape=jax.ShapeDtypeStruct((R4, H), jnp.float32),
)


def _scale_body(x_ref, d_ref, o_ref):
    o_ref[...] = d_ref[...] * x_ref[...]


_scale_call = pl.pallas_call(
    _scale_body,
    grid=(R4 // _RB,),
    in_specs=[
        pl.BlockSpec((_RB, H), lambda i: (i, 0)),
        pl.BlockSpec((_RB, 1), lambda i: (i, 0)),
    ],
    out_specs=pl.BlockSpec((_RB, H), lambda i: (i, 0)),
    out_shape=jax.ShapeDtypeStruct((R4, H), jnp.float32),
)


def _comb_body(p_ref, g_ref, d_ref, b_ref, w_ref, h_ref, gn_ref):
    d = d_ref[...]
    h = _elu(d * (p_ref[0] + p_ref[1] + g_ref[...]) + b_ref[...])
    h_ref[...] = h
    gn_ref[...] = d * jnp.dot(h, w_ref[...], preferred_element_type=jnp.float32)


_comb_call = pl.pallas_call(
    _comb_body,
    grid=(R4 // _RB,),
    in_specs=[
        pl.BlockSpec((NCORE, _RB, H), lambda i: (0, i, 0)),
        pl.BlockSpec((_RB, H), lambda i: (i, 0)),
        pl.BlockSpec((_RB, 1), lambda i: (i, 0)),
        pl.BlockSpec((1, H), lambda i: (0, 0)),
        pl.BlockSpec((H, H), lambda i: (0, 0)),
    ],
    out_specs=[
        pl.BlockSpec((_RB, H), lambda i: (i, 0)),
        pl.BlockSpec((_RB, H), lambda i: (i, 0)),
    ],
    out_shape=[
        jax.ShapeDtypeStruct((R4, H), jnp.float32),
        jax.ShapeDtypeStruct((R4, H), jnp.float32),
    ],
)


def _xg_body(p_ref, g_ref, d_ref, b_ref, h1_ref, h2_ref, wf_ref, bf_ref, xg_ref):
    d = d_ref[...]
    h3 = _elu(d * (p_ref[0] + p_ref[1] + g_ref[...]) + b_ref[...])
    wf = wf_ref[...]
    xg_ref[...] = (
        jnp.dot(h1_ref[...], wf[:, 0:1], preferred_element_type=jnp.float32)
        + jnp.dot(h2_ref[...], wf[:, 1:2], preferred_element_type=jnp.float32)
        + jnp.dot(h3, wf[:, 2:3], preferred_element_type=jnp.float32)
        + bf_ref[...])


_xg_call = pl.pallas_call(
    _xg_body,
    grid=(R4 // _RB,),
    in_specs=[
        pl.BlockSpec((NCORE, _RB, H), lambda i: (0, i, 0)),
        pl.BlockSpec((_RB, H), lambda i: (i, 0)),
        pl.BlockSpec((_RB, 1), lambda i: (i, 0)),
        pl.BlockSpec((1, H), lambda i: (0, 0)),
        pl.BlockSpec((_RB, H), lambda i: (i, 0)),
        pl.BlockSpec((_RB, H), lambda i: (i, 0)),
        pl.BlockSpec((H, 3), lambda i: (0, 0)),
        pl.BlockSpec((1, 1), lambda i: (0, 0)),
    ],
    out_specs=pl.BlockSpec((_RB, 1), lambda i: (i, 0)),
    out_shape=jax.ShapeDtypeStruct((R4, 1), jnp.float32),
)

_HKB = 1536
_HSTEPS = NP // _HKB


def _head_body(xg_ref, wl1_ref, bl1_ref, wl2_ref, bl2_ref, out_ref, acc_ref):
    k = pl.program_id(0)

    @pl.when(k == 0)
    def _init():
        acc_ref[...] = jnp.zeros_like(acc_ref)

    acc_ref[...] += jnp.dot(
        xg_ref[...], wl1_ref[...], preferred_element_type=jnp.float32)

    @pl.when(k == _HSTEPS - 1)
    def _fin():
        y = _elu(acc_ref[...] + bl1_ref[...])
        z = jnp.dot(y, wl2_ref[...], preferred_element_type=jnp.float32)
        z = z + bl2_ref[...]
        m = jnp.max(z, axis=-1, keepdims=True)
        ez = jnp.exp(z - m)
        out_ref[...] = (z - m) - jnp.log(jnp.sum(ez, axis=-1, keepdims=True))


_head_call = pl.pallas_call(
    _head_body,
    grid=(_HSTEPS,),
    in_specs=[
        pl.BlockSpec((8, _HKB), lambda k: (0, k)),
        pl.BlockSpec((_HKB, 256), lambda k: (k, 0)),
        pl.BlockSpec((1, 256), lambda k: (0, 0)),
        pl.BlockSpec((256, 128), lambda k: (0, 0)),
        pl.BlockSpec((1, 128), lambda k: (0, 0)),
    ],
    out_specs=pl.BlockSpec((8, 128), lambda k: (0, 0)),
    out_shape=jax.ShapeDtypeStruct((8, 128), jnp.float32),
    scratch_shapes=[pltpu.VMEM((8, 256), jnp.float32)],
)


def kernel(x, batch, edge_index, nodes_graphlets,
           W1, b1, W2, b2, W3, b3, Wfc, bfc, Wl1, bl1, Wl2, bl2):
    f32 = jnp.float32
    G = nodes_graphlets.shape[1]

    xt = x.reshape(B, N).T
    xpb = xt.reshape(N, 2, 2).transpose(1, 0, 2)
    pe = jnp.broadcast_to(nodes_graphlets[None, :, None, :], (2, N, 2, G))
    h0 = jnp.concatenate(
        [xpb[..., None], pe, jnp.zeros((2, N, 2, 128 - 1 - G), f32)], axis=3)
    h0 = jnp.pad(h0, ((0, 0), (0, NP - N), (0, 0), (0, 0))).reshape(R4, 128)
    W1p = jnp.pad(W1, ((0, 128 - 1 - G), (0, 0)))

    src = jnp.pad(edge_index[0], (0, EP - E),
                  constant_values=N).reshape(NW * NCHUNK, CH)
    dst = jnp.pad(edge_index[1], (0, EP - E),
                  constant_values=N).reshape(NW * NCHUNK, CH)

    _sc_hist, _sc_scatter = _sc_kernels()
    hist = _sc_hist(dst)
    dinv = _dinv_call(hist)
    dnb = jnp.broadcast_to(dinv.reshape(NP, 1), (NP, 2)).reshape(NP * 2, 1)
    d4 = jnp.concatenate([dnb, dnb], axis=0)

    def scat(g):
        return _sc_scatter(
            g.reshape(2, NP, NB // 2), src, dst).reshape(NCORE, R4, H)

    z1 = _mm_call(h0, W1p)
    g1 = _scale_call(z1, d4)
    p1 = scat(g1)
    h1, g2 = _comb_call(p1, g1, d4, b1[None], W2)
    p2 = scat(g2)
    h2, g3 = _comb_call(p2, g2, d4, b2[None], W3)
    p3 = scat(g3)

    xg = _xg_call(p3, g3, d4, b3[None], h1, h2,
                  Wfc.reshape(H, 3), bfc.reshape(1, 1))

    xgt = xg.reshape(2, NP, 2).transpose(1, 0, 2).reshape(NP, B)[:N].T
    xg8 = jnp.zeros((8, NP), f32).at[:B, :N].set(xgt)
    Wl1p = jnp.pad(Wl1, ((0, NP - N), (0, 0)))
    Wl2p = jnp.pad(Wl2, ((0, 0), (0, 126)))
    bl2p = jnp.concatenate([bl2, jnp.full((126,), -1e30, f32)])

    out = _head_call(xg8, Wl1p, bl1[None], Wl2p, bl2p[None])
    return out[:B, :2]


# speedup vs baseline: 95.3533x; 1.1194x over previous
import functools

import jax
import jax.numpy as jnp
from jax import lax
from jax.experimental import pallas as pl
from jax.experimental.pallas import tpu as pltpu
from jax.experimental.pallas import tpu_sc as plsc

N = 15135
NP = 15360
B = 4
H = 32
NB = B * H
R4 = NP * B
E = 484320
NCORE = 2
NSUB = 16
NW = NCORE * NSUB
CH = 128
NCHUNK = 120
EW = CH * NCHUNK
EP = NW * EW
RT = NP // NSUB
ZR = 60
NBUF = 4
NGRP = NCHUNK // NBUF

def _elu(v):
    return jnp.where(v > 0, v, jnp.exp(jnp.minimum(v, 0.0)) - 1.0)


def _sc_hist_body(dst_hbm, hist_hbm, idx_v, h0_v, h1_v, h2_v, h3_v):
    c = lax.axis_index("c")
    s = lax.axis_index("s")
    w = c * NSUB + s
    hists = [h0_v, h1_v, h2_v, h3_v]

    pltpu.sync_copy(dst_hbm.at[pl.ds(w * NCHUNK, NCHUNK)], idx_v)

    zeros16 = jnp.zeros((16,), jnp.float32)

    def zero_body(i, carry):
        for j in range(8):
            sl = pl.ds(pl.multiple_of(i * 128 + j * 16, 16), 16)
            for hv in hists:
                hv[sl] = zeros16
        return carry

    lax.fori_loop(0, NP // 128, zero_body, 0)

    ones = jnp.ones((16,), jnp.float32)

    def chunk_body(i, carry):
        for j in range(CH // 16):
            idx16 = idx_v[i, pl.ds(pl.multiple_of(j * 16, 16), 16)]
            plsc.addupdate_scatter(hists[j % 4], [idx16], ones)
        return carry

    lax.fori_loop(0, NCHUNK, chunk_body, 0)
    for k, hv in enumerate(hists):
        pltpu.sync_copy(hv, hist_hbm.at[w, k])


def _sc_scatter_body(g_hbm, src_hbm, dst_hbm, part_hbm,
                     idxs_v, idxd_v, rows_v, zero_v, acc_sh, *sems):
    gsems = sems[:NBUF]
    ssems = sems[NBUF:]
    c = lax.axis_index("c")
    s = lax.axis_index("s")
    w = c * NSUB + s

    pltpu.sync_copy(src_hbm.at[pl.ds(w * NCHUNK, NCHUNK)], idxs_v)
    pltpu.sync_copy(dst_hbm.at[pl.ds(w * NCHUNK, NCHUNK)], idxd_v)

    def zfill(i, carry):
        zero_v[i // 4, pl.ds(pl.multiple_of((i % 4) * 16, 16), 16)] = (
            jnp.zeros((16,), jnp.float32))
        return carry

    lax.fori_loop(0, ZR * 4, zfill, 0)

    for half in range(2):
        def zcopy(j, carry):
            pltpu.sync_copy(zero_v, acc_sh.at[pl.ds(s * RT + j * ZR, ZR)])
            return carry

        lax.fori_loop(0, RT // ZR, zcopy, 0)
        plsc.subcore_barrier()

        g_half = g_hbm.at[half]

        def gather(ci, b):
            pltpu.async_copy(
                g_half.at[idxs_v.at[ci]], rows_v.at[b], gsems[b])

        def gwait(b):
            pltpu.make_async_copy(
                g_half.at[idxs_v.at[0]], rows_v.at[b], gsems[b]).wait()

        def scat(ci, b):
            pltpu.async_copy(
                rows_v.at[b], acc_sh.at[idxd_v.at[ci]], ssems[b], add=True)

        def swait(b):
            pltpu.make_async_copy(
                rows_v.at[b], acc_sh.at[idxd_v.at[0]], ssems[b]).wait()

        for b in range(NBUF):
            gather(b, b)

        def group_body(m, carry):
            base = m * NBUF
            gwait(0); scat(base + 0, 0)
            gwait(1); scat(base + 1, 1)
            swait(0); gather(base + 4, 0)
            swait(1); gather(base + 5, 1)
            gwait(2); scat(base + 2, 2)
            gwait(3); scat(base + 3, 3)
            swait(2); gather(base + 6, 2)
            swait(3); gather(base + 7, 3)
            return carry

        lax.fori_loop(0, NGRP - 1, group_body, 0)
        for b in range(NBUF):
            gwait(b)
            scat(NCHUNK - NBUF + b, b)
        for b in range(NBUF):
            swait(b)
        plsc.subcore_barrier()

        ro = pl.multiple_of(s * RT, 8)
        pltpu.sync_copy(acc_sh.at[pl.ds(ro, RT)],
                        part_hbm.at[c, half, pl.ds(ro, RT)])
        plsc.subcore_barrier()


@functools.lru_cache(maxsize=1)
def _sc_kernels():
    mesh = plsc.VectorSubcoreMesh(
        core_axis_name="c", subcore_axis_name="s",
        num_cores=NCORE, num_subcores=NSUB)
    params = pltpu.CompilerParams(
        needs_layout_passes=False, use_tc_tiling_on_sc=False)
    sc_hist = pl.kernel(
        _sc_hist_body,
        out_type=jax.ShapeDtypeStruct((NW, 4, NP), jnp.float32),
        mesh=mesh,
        compiler_params=params,
        scratch_types=[
            pltpu.VMEM((NCHUNK, CH), jnp.int32),
            pltpu.VMEM((NP,), jnp.float32),
            pltpu.VMEM((NP,), jnp.float32),
            pltpu.VMEM((NP,), jnp.float32),
            pltpu.VMEM((NP,), jnp.float32),
        ],
    )
    sc_scatter = pl.kernel(
        _sc_scatter_body,
        out_type=jax.ShapeDtypeStruct((NCORE, 2, NP, NB // 2), jnp.float32),
        mesh=mesh,
        compiler_params=params,
        scratch_types=(
            [
                pltpu.VMEM((NCHUNK, CH), jnp.int32),
                pltpu.VMEM((NCHUNK, CH), jnp.int32),
                pltpu.VMEM((NBUF, CH, NB // 2), jnp.float32),
                pltpu.VMEM((ZR, NB // 2), jnp.float32),
                pltpu.VMEM_SHARED((NP, NB // 2), jnp.float32),
            ]
            + [pltpu.SemaphoreType.DMA] * (2 * NBUF)
        ),
    )
    return sc_hist, sc_scatter


_DKB = 1536


def _dinv_body(hist_ref, out_ref):
    deg = jnp.sum(hist_ref[...], axis=0, keepdims=True) + 1.0
    out_ref[...] = lax.rsqrt(deg)


_dinv_call = pl.pallas_call(
    _dinv_body,
    grid=(NP // _DKB,),
    in_specs=[pl.BlockSpec((NW * 4, _DKB), lambda i: (0, i))],
    out_specs=pl.BlockSpec((1, _DKB), lambda i: (0, i)),
    out_shape=jax.ShapeDtypeStruct((1, NP), jnp.float32),
)

_RB = 4096


def _mm_body(x_ref, w_ref, o_ref):
    o_ref[...] = jnp.dot(
        x_ref[...], w_ref[...], preferred_element_type=jnp.float32)


_mm_call = pl.pallas_call(
    _mm_body,
    grid=(R4 // _RB,),
    in_specs=[
        pl.BlockSpec((_RB, 128), lambda i: (i, 0)),
        pl.BlockSpec((128, H), lambda i: (0, 0)),
    ],
    out_specs=pl.BlockSpec((_RB, H), lambda i: (i, 0)),
    out_shape=jax.ShapeDtypeStruct((R4, H), jnp.float32),
)


def _scale_body(x_ref, d_ref, o_ref):
    o_ref[...] = d_ref[...] * x_ref[...]


_scale_call = pl.pallas_call(
    _scale_body,
    grid=(R4 // _RB,),
    in_specs=[
        pl.BlockSpec((_RB, H), lambda i: (i, 0)),
        pl.BlockSpec((_RB, 1), lambda i: (i, 0)),
    ],
    out_specs=pl.BlockSpec((_RB, H), lambda i: (i, 0)),
    out_shape=jax.ShapeDtypeStruct((R4, H), jnp.float32),
)


_NB2 = NB // 2
_CBK = 1536


def _combine128(p_ref, g_ref, d_ref, b_ref):
    pc = jnp.concatenate(
        [p_ref[0, 0] + p_ref[1, 0] + g_ref[0],
         p_ref[0, 1] + p_ref[1, 1] + g_ref[1]], axis=-1)
    return _elu(d_ref[...] * pc + b_ref[...])


def _comb_body(p_ref, g_ref, d_ref, b_ref, w_ref, h_ref, gn_ref):
    h = _combine128(p_ref, g_ref, d_ref, b_ref)
    h_ref[...] = h
    gn = d_ref[...] * jnp.dot(
        h, w_ref[...], preferred_element_type=jnp.float32)
    gn_ref[0] = gn[:, :_NB2]
    gn_ref[1] = gn[:, _NB2:]


_comb_call = pl.pallas_call(
    _comb_body,
    grid=(NP // _CBK,),
    in_specs=[
        pl.BlockSpec((NCORE, 2, _CBK, _NB2), lambda i: (0, 0, i, 0)),
        pl.BlockSpec((2, _CBK, _NB2), lambda i: (0, i, 0)),
        pl.BlockSpec((_CBK, 1), lambda i: (i, 0)),
        pl.BlockSpec((1, NB), lambda i: (0, 0)),
        pl.BlockSpec((NB, NB), lambda i: (0, 0)),
    ],
    out_specs=[
        pl.BlockSpec((_CBK, NB), lambda i: (i, 0)),
        pl.BlockSpec((2, _CBK, _NB2), lambda i: (0, i, 0)),
    ],
    out_shape=[
        jax.ShapeDtypeStruct((NP, NB), jnp.float32),
        jax.ShapeDtypeStruct((2, NP, _NB2), jnp.float32),
    ],
)


def _xg_body(p_ref, g_ref, d_ref, b_ref, h1_ref, h2_ref,
             w1_ref, w2_ref, w3_ref, bf_ref, xg_ref):
    h3 = _combine128(p_ref, g_ref, d_ref, b_ref)
    xg_ref[...] = (
        jnp.dot(h1_ref[...], w1_ref[...], preferred_element_type=jnp.float32)
        + jnp.dot(h2_ref[...], w2_ref[...], preferred_element_type=jnp.float32)
        + jnp.dot(h3, w3_ref[...], preferred_element_type=jnp.float32)
        + bf_ref[...])


_xg_call = pl.pallas_call(
    _xg_body,
    grid=(NP // _CBK,),
    in_specs=[
        pl.BlockSpec((NCORE, 2, _CBK, _NB2), lambda i: (0, 0, i, 0)),
        pl.BlockSpec((2, _CBK, _NB2), lambda i: (0, i, 0)),
        pl.BlockSpec((_CBK, 1), lambda i: (i, 0)),
        pl.BlockSpec((1, NB), lambda i: (0, 0)),
        pl.BlockSpec((_CBK, NB), lambda i: (i, 0)),
        pl.BlockSpec((_CBK, NB), lambda i: (i, 0)),
        pl.BlockSpec((NB, B), lambda i: (0, 0)),
        pl.BlockSpec((NB, B), lambda i: (0, 0)),
        pl.BlockSpec((NB, B), lambda i: (0, 0)),
        pl.BlockSpec((1, 1), lambda i: (0, 0)),
    ],
    out_specs=pl.BlockSpec((_CBK, B), lambda i: (i, 0)),
    out_shape=jax.ShapeDtypeStruct((NP, B), jnp.float32),
)

_HKB = 1536
_HSTEPS = NP // _HKB


def _head_body(xg_ref, wl1_ref, bl1_ref, wl2_ref, bl2_ref, out_ref, acc_ref):
    k = pl.program_id(0)

    @pl.when(k == 0)
    def _init():
        acc_ref[...] = jnp.zeros_like(acc_ref)

    acc_ref[...] += jnp.dot(
        xg_ref[...], wl1_ref[...], preferred_element_type=jnp.float32)

    @pl.when(k == _HSTEPS - 1)
    def _fin():
        y = _elu(acc_ref[...] + bl1_ref[...])
        z = jnp.dot(y, wl2_ref[...], preferred_element_type=jnp.float32)
        z = z + bl2_ref[...]
        m = jnp.max(z, axis=-1, keepdims=True)
        ez = jnp.exp(z - m)
        out_ref[...] = (z - m) - jnp.log(jnp.sum(ez, axis=-1, keepdims=True))


_head_call = pl.pallas_call(
    _head_body,
    grid=(_HSTEPS,),
    in_specs=[
        pl.BlockSpec((8, _HKB), lambda k: (0, k)),
        pl.BlockSpec((_HKB, 256), lambda k: (k, 0)),
        pl.BlockSpec((1, 256), lambda k: (0, 0)),
        pl.BlockSpec((256, 128), lambda k: (0, 0)),
        pl.BlockSpec((1, 128), lambda k: (0, 0)),
    ],
    out_specs=pl.BlockSpec((8, 128), lambda k: (0, 0)),
    out_shape=jax.ShapeDtypeStruct((8, 128), jnp.float32),
    scratch_shapes=[pltpu.VMEM((8, 256), jnp.float32)],
)


def kernel(x, batch, edge_index, nodes_graphlets,
           W1, b1, W2, b2, W3, b3, Wfc, bfc, Wl1, bl1, Wl2, bl2):
    f32 = jnp.float32
    G = nodes_graphlets.shape[1]

    xt = x.reshape(B, N).T
    xpb = xt.reshape(N, 2, 2).transpose(1, 0, 2)
    pe = jnp.broadcast_to(nodes_graphlets[None, :, None, :], (2, N, 2, G))
    h0 = jnp.concatenate(
        [xpb[..., None], pe, jnp.zeros((2, N, 2, 128 - 1 - G), f32)], axis=3)
    h0 = jnp.pad(h0, ((0, 0), (0, NP - N), (0, 0), (0, 0))).reshape(R4, 128)
    W1p = jnp.pad(W1, ((0, 128 - 1 - G), (0, 0)))

    src = jnp.pad(edge_index[0], (0, EP - E),
                  constant_values=N).reshape(NW * NCHUNK, CH)
    dst = jnp.pad(edge_index[1], (0, EP - E),
                  constant_values=N).reshape(NW * NCHUNK, CH)

    _sc_hist, _sc_scatter = _sc_kernels()
    hist = _sc_hist(dst).reshape(NW * 4, NP)
    dinv = _dinv_call(hist)
    d1 = dinv.reshape(NP, 1)
    dnb = jnp.broadcast_to(d1, (NP, 2)).reshape(NP * 2, 1)
    d4 = jnp.concatenate([dnb, dnb], axis=0)

    eye4 = jnp.eye(B, dtype=f32)
    W4_2 = jnp.kron(eye4, W2)
    W4_3 = jnp.kron(eye4, W3)
    wf = Wfc.reshape(H, 3)
    Wb1 = jnp.kron(eye4, wf[:, 0:1])
    Wb2 = jnp.kron(eye4, wf[:, 1:2])
    Wb3 = jnp.kron(eye4, wf[:, 2:3])

    z1 = _mm_call(h0, W1p)
    g1 = _scale_call(z1, d4).reshape(2, NP, NB // 2)
    p1 = _sc_scatter(g1, src, dst)
    h1, g2 = _comb_call(p1, g1, d1, jnp.tile(b1, B)[None], W4_2)
    p2 = _sc_scatter(g2, src, dst)
    h2, g3 = _comb_call(p2, g2, d1, jnp.tile(b2, B)[None], W4_3)
    p3 = _sc_scatter(g3, src, dst)

    xg = _xg_call(p3, g3, d1, jnp.tile(b3, B)[None], h1, h2,
                  Wb1, Wb2, Wb3, bfc.reshape(1, 1))

    xgt = xg[:N].T
    xg8 = jnp.zeros((8, NP), f32).at[:B, :N].set(xgt)
    Wl1p = jnp.pad(Wl1, ((0, NP - N), (0, 0)))
    Wl2p = jnp.pad(Wl2, ((0, 0), (0, 126)))
    bl2p = jnp.concatenate([bl2, jnp.full((126,), -1e30, f32)])

    out = _head_call(xg8, Wl1p, bl1[None], Wl2p, bl2p[None])
    return out[:B, :2]
